# Initial kernel scaffold; baseline (speedup 1.0000x reference)
import functools

import jax
import jax.numpy as jnp
from jax import lax
from jax.experimental import pallas as pl
from jax.experimental.pallas import tpu as pltpu
from jax.experimental.pallas import tpu_sc as plsc

_N = 100000
_E = 1600000
_ED = 16
_D = 128

_NC = 2
_NS = 16
_NW = _NC * _NS

_CH = 128
_ROWS = _E // _CH
_BASE = _ROWS // _NW
_EXTRA = _ROWS % _NW
_K = 13
_NSLABS = _BASE // _K
_NSLICE = _N // _NS

assert _K * _NSLABS == _BASE
assert _NSLABS % 2 == 0


def _sc_body(row_hbm, attr_hbm, zsum_hbm, zcnt_hbm, ones_hbm,
             psum_hbm, pcnt_hbm,
             sum_sh, cnt_sh, idx_v, attr_v, ones_v,
             sem_i0, sem_i1, sem_a0, sem_a1, sem_s, sem_c):
  cid = lax.axis_index("c")
  sid = lax.axis_index("s")
  wid = sid * _NC + cid

  sem_i = (sem_i0, sem_i1)
  sem_a = (sem_a0, sem_a1)

  n0 = sid * _NSLICE
  pltpu.sync_copy(zsum_hbm, sum_sh.at[pl.ds(n0, _NSLICE), :])
  pltpu.sync_copy(zcnt_hbm, cnt_sh.at[pl.ds(n0, _NSLICE), :])
  pltpu.sync_copy(ones_hbm, ones_v)
  plsc.subcore_barrier()

  base = wid * _BASE + jnp.minimum(wid, _EXTRA)

  def _fetch(slab, b):
    r = base + slab * _K
    pltpu.async_copy(row_hbm.at[pl.ds(r, _K), :], idx_v.at[b], sem_i[b])
    pltpu.async_copy(attr_hbm.at[pl.ds(r * _CH, _K * _CH), :], attr_v.at[b],
                     sem_a[b])

  def _wait_fetch(slab, b):
    r = base + slab * _K
    pltpu.make_async_copy(row_hbm.at[pl.ds(r, _K), :], idx_v.at[b],
                          sem_i[b]).wait()
    pltpu.make_async_copy(attr_hbm.at[pl.ds(r * _CH, _K * _CH), :],
                          attr_v.at[b], sem_a[b]).wait()

  def _scatter_slab(b):
    handles = []
    for k in range(_K):
      idx = idx_v.at[b, k]
      hs = pltpu.async_copy(attr_v.at[b, pl.ds(k * _CH, _CH), :],
                            sum_sh.at[idx], sem_s, add=True)
      hc = pltpu.async_copy(ones_v, cnt_sh.at[idx], sem_c, add=True)
      handles.append((hs, hc))
    for hs, hc in handles:
      hs.wait()
      hc.wait()

  _fetch(0, 0)

  def _loop_body(i, carry):
    s0 = i * 2
    _wait_fetch(s0, 0)
    _fetch(s0 + 1, 1)
    _scatter_slab(0)
    _wait_fetch(s0 + 1, 1)

    @pl.when(i < _NSLABS // 2 - 1)
    def _():
      _fetch(s0 + 2, 0)

    _scatter_slab(1)
    return carry

  lax.fori_loop(0, _NSLABS // 2, _loop_body, 0)

  @pl.when(wid < _EXTRA)
  def _():
    r = base + _BASE
    pltpu.sync_copy(row_hbm.at[pl.ds(r, 1), :], idx_v.at[0, pl.ds(0, 1), :])
    pltpu.sync_copy(attr_hbm.at[pl.ds(r * _CH, _CH), :],
                    attr_v.at[0, pl.ds(0, _CH), :])
    idx = idx_v.at[0, 0]
    pltpu.sync_copy(attr_v.at[0, pl.ds(0, _CH), :], sum_sh.at[idx], add=True)
    pltpu.sync_copy(ones_v, cnt_sh.at[idx], add=True)

  plsc.subcore_barrier()

  pltpu.sync_copy(sum_sh.at[pl.ds(n0, _NSLICE), :],
                  psum_hbm.at[cid, pl.ds(n0, _NSLICE), :])
  pltpu.sync_copy(cnt_sh.at[pl.ds(n0, _NSLICE), :],
                  pcnt_hbm.at[cid, pl.ds(n0, _NSLICE), :])


_segment_sums = functools.partial(
    pl.kernel,
    mesh=plsc.VectorSubcoreMesh(core_axis_name="c", subcore_axis_name="s"),
    out_type=[
        jax.ShapeDtypeStruct((_NC, _N, _ED), jnp.float32),
        jax.ShapeDtypeStruct((_NC, _N, 1), jnp.float32),
    ],
    scratch_types=[
        pltpu.VMEM_SHARED((_N, _ED), jnp.float32),
        pltpu.VMEM_SHARED((_N, 1), jnp.float32),
        pltpu.VMEM((2, _K, _CH), jnp.int32),
        pltpu.VMEM((2, _K * _CH, _ED), jnp.float32),
        pltpu.VMEM((_CH, 1), jnp.float32),
        pltpu.SemaphoreType.DMA,
        pltpu.SemaphoreType.DMA,
        pltpu.SemaphoreType.DMA,
        pltpu.SemaphoreType.DMA,
        pltpu.SemaphoreType.DMA,
        pltpu.SemaphoreType.DMA,
    ],
)(_sc_body)


_BN = 2000
_NBLK = _N // _BN


def _mlp_body(x_ref, s0_ref, s1_ref, c0_ref, c1_ref,
              w1x_ref, w1e_ref, b1_ref, w2_ref, b2_ref, wa_ref, ba_ref,
              w3x_ref, w3h_ref, b3_ref, w4_ref, b4_ref, o_ref):
  f32 = jnp.float32
  x = x_ref[...]
  s = s0_ref[...] + s1_ref[...]
  c = c0_ref[...] + c1_ref[...]
  agg = s / jnp.maximum(c, 1.0)
  h = (jnp.dot(x, w1x_ref[...], preferred_element_type=f32)
       + jnp.dot(agg, w1e_ref[...], preferred_element_type=f32)
       + b1_ref[...])
  h = jnp.maximum(h, 0.0)
  h = jnp.dot(h, w2_ref[...], preferred_element_type=f32) + b2_ref[...]
  att = jax.nn.sigmoid(
      jnp.dot(h, wa_ref[...], preferred_element_type=f32) + ba_ref[...])
  h = h * att
  h = jnp.maximum(
      jnp.dot(x, w3x_ref[...], preferred_element_type=f32)
      + jnp.dot(h, w3h_ref[...], preferred_element_type=f32)
      + b3_ref[...], 0.0)
  o_ref[...] = jnp.dot(h, w4_ref[...], preferred_element_type=f32) + b4_ref[...]


def _row_spec(c):
  return pl.BlockSpec((_BN, c), lambda i: (i, 0))


def _const_spec(shape):
  return pl.BlockSpec(shape, lambda i: (0, 0))


_mlp = pl.pallas_call(
    _mlp_body,
    grid=(_NBLK,),
    in_specs=[
        _row_spec(_D),
        _row_spec(_ED),
        _row_spec(_ED),
        _row_spec(1),
        _row_spec(1),
        _const_spec((_D, _D)),
        _const_spec((_ED, _D)),
        _const_spec((1, _D)),
        _const_spec((_D, _D)),
        _const_spec((1, _D)),
        _const_spec((_D, 1)),
        _const_spec((1, 1)),
        _const_spec((_D, _D)),
        _const_spec((_D, _D)),
        _const_spec((1, _D)),
        _const_spec((_D, _D)),
        _const_spec((1, _D)),
    ],
    out_specs=_row_spec(_D),
    out_shape=jax.ShapeDtypeStruct((_N, _D), jnp.float32),
    compiler_params=pltpu.CompilerParams(dimension_semantics=("arbitrary",)),
)


def kernel(x, edge_index, edge_attr, u, batch,
           W1, b1, W2, b2, Wa, ba, W3, b3, W4, b4):
  del u, batch
  row = edge_index[0].astype(jnp.int32).reshape(_ROWS, _CH)
  zsum = jnp.zeros((_NSLICE, _ED), jnp.float32)
  zcnt = jnp.zeros((_NSLICE, 1), jnp.float32)
  ones = jnp.ones((_CH, 1), jnp.float32)
  psum, pcnt = _segment_sums(row, edge_attr, zsum, zcnt, ones)
  return _mlp(x, psum[0], psum[1], pcnt[0], pcnt[1],
              W1[:_D], W1[_D:], b1.reshape(1, _D),
              W2, b2.reshape(1, _D),
              Wa, ba.reshape(1, 1),
              W3[:_D], W3[_D:], b3.reshape(1, _D),
              W4, b4.reshape(1, _D))


# speedup vs baseline: 6.6418x; 6.6418x over previous
import functools

import jax
import jax.numpy as jnp
from jax import lax
from jax.experimental import pallas as pl
from jax.experimental.pallas import tpu as pltpu
from jax.experimental.pallas import tpu_sc as plsc

_N = 100000
_E = 1600000
_ED = 16
_D = 128

_NC = 2
_NS = 16

_CH = 128
_ROWS = _E // _CH
_NDUM = 8
_RPAD = 12544
_BASE = _RPAD // _NS
_K = 4
_NSLABS = _BASE // _K

_ZSTART = 6240
_ZLEN = 6400
_CPY = 320

assert _K * _NSLABS == _BASE and _NSLABS % 2 == 0
assert (_NS - 1) * _ZSTART + _ZLEN == _N
assert _ZLEN % _CPY == 0 and _ZSTART % 8 == 0 and _CPY % 8 == 0
assert _CPY <= _K * _CH


def _sc_body(row_hbm, attr_hbm, zero_hbm, ones_hbm,
             acc_out,
             acc_sh, idx_v, attr_v,
             sem_i0, sem_i1, sem_a0, sem_a1, sem_s):
  cid = lax.axis_index("c")
  sid = lax.axis_index("s")

  sem_i = (sem_i0, sem_i1)
  sem_a = (sem_a0, sem_a1)

  n0 = sid * _ZSTART
  pltpu.sync_copy(zero_hbm, attr_v.at[0, pl.ds(0, _CPY), :])
  for j in range(_ZLEN // _CPY):
    pltpu.sync_copy(attr_v.at[0, pl.ds(0, _CPY), :],
                    acc_sh.at[pl.ds(n0 + j * _CPY, _CPY), :])

  @pl.when(cid == 1)
  def _():
    pltpu.sync_copy(ones_hbm, attr_v.at[0])
    pltpu.sync_copy(ones_hbm, attr_v.at[1])

  plsc.subcore_barrier()

  base = sid * _BASE

  def _fetch(slab, b):
    r = base + slab * _K
    ra = jnp.minimum(r, _ROWS - _K)
    pltpu.async_copy(row_hbm.at[pl.ds(r, _K), :, :], idx_v.at[b], sem_i[b])

    @pl.when(cid == 0)
    def _():
      pltpu.async_copy(attr_hbm.at[pl.ds(ra * _CH, _K * _CH), :],
                       attr_v.at[b], sem_a[b])

  def _wait_fetch(slab, b):
    r = base + slab * _K
    ra = jnp.minimum(r, _ROWS - _K)
    pltpu.make_async_copy(row_hbm.at[pl.ds(r, _K), :, :], idx_v.at[b],
                          sem_i[b]).wait()

    @pl.when(cid == 0)
    def _():
      pltpu.make_async_copy(attr_hbm.at[pl.ds(ra * _CH, _K * _CH), :],
                            attr_v.at[b], sem_a[b]).wait()

  def _scatter_slab(b):
    handles = []
    for k in range(_K):
      idx = idx_v.at[b, k, 0]
      handles.append(
          pltpu.async_copy(attr_v.at[b, pl.ds(k * _CH, _CH), :],
                           acc_sh.at[idx], sem_s, add=True))
    for hs in handles:
      hs.wait()

  _fetch(0, 0)

  def _loop_body(i, carry):
    s0 = i * 2
    _wait_fetch(s0, 0)
    _fetch(s0 + 1, 1)
    _scatter_slab(0)
    _wait_fetch(s0 + 1, 1)
    _fetch(s0 + 2, 0)
    _scatter_slab(1)
    return carry

  lax.fori_loop(0, _NSLABS // 2 - 1, _loop_body, 0)

  _wait_fetch(_NSLABS - 2, 0)
  _fetch(_NSLABS - 1, 1)
  _scatter_slab(0)
  _wait_fetch(_NSLABS - 1, 1)
  _scatter_slab(1)

  plsc.subcore_barrier()

  for j in range(_ZLEN // _CPY):
    o = n0 + j * _CPY
    pltpu.sync_copy(acc_sh.at[pl.ds(o, _CPY), :],
                    attr_v.at[0, pl.ds(0, _CPY), :])
    pltpu.sync_copy(attr_v.at[0, pl.ds(0, _CPY), :],
                    acc_out.at[cid, pl.ds(o, _CPY), :])


_segment_sums = functools.partial(
    pl.kernel,
    mesh=plsc.VectorSubcoreMesh(core_axis_name="c", subcore_axis_name="s"),
    out_type=jax.ShapeDtypeStruct((_NC, _N, _ED), jnp.float32),
    scratch_types=[
        pltpu.VMEM_SHARED((_N + _NDUM, _ED), jnp.float32),
        pltpu.VMEM((2, _K, 1, _CH), jnp.int32),
        pltpu.VMEM((2, _K * _CH, _ED), jnp.float32),
        pltpu.SemaphoreType.DMA,
        pltpu.SemaphoreType.DMA,
        pltpu.SemaphoreType.DMA,
        pltpu.SemaphoreType.DMA,
        pltpu.SemaphoreType.DMA,
    ],
    compiler_params=pltpu.CompilerParams(use_tc_tiling_on_sc=False),
)(_sc_body)


_BN = 2000
_NBLK = _N // _BN


def _mlp_body(x_ref, s_ref, c_ref,
              w1x_ref, w1e_ref, b1_ref, w2_ref, b2_ref, wa_ref, ba_ref,
              w3x_ref, w3h_ref, b3_ref, w4_ref, b4_ref, o_ref):
  f32 = jnp.float32
  x = x_ref[...]
  s = s_ref[...]
  c = c_ref[:, 0:1]
  agg = s / jnp.maximum(c, 1.0)
  h = (jnp.dot(x, w1x_ref[...], preferred_element_type=f32)
       + jnp.dot(agg, w1e_ref[...], preferred_element_type=f32)
       + b1_ref[...])
  h = jnp.maximum(h, 0.0)
  h = jnp.dot(h, w2_ref[...], preferred_element_type=f32) + b2_ref[...]
  att = jax.nn.sigmoid(
      jnp.dot(h, wa_ref[...], preferred_element_type=f32) + ba_ref[...])
  h = h * att
  h = jnp.maximum(
      jnp.dot(x, w3x_ref[...], preferred_element_type=f32)
      + jnp.dot(h, w3h_ref[...], preferred_element_type=f32)
      + b3_ref[...], 0.0)
  o_ref[...] = jnp.dot(h, w4_ref[...], preferred_element_type=f32) + b4_ref[...]


def _row_spec(c):
  return pl.BlockSpec((_BN, c), lambda i: (i, 0))


def _const_spec(shape):
  return pl.BlockSpec(shape, lambda i: (0, 0))


_mlp = pl.pallas_call(
    _mlp_body,
    grid=(_NBLK,),
    in_specs=[
        _row_spec(_D),
        _row_spec(_ED),
        _row_spec(_ED),
        _const_spec((_D, _D)),
        _const_spec((_ED, _D)),
        _const_spec((1, _D)),
        _const_spec((_D, _D)),
        _const_spec((1, _D)),
        _const_spec((_D, 1)),
        _const_spec((1, 1)),
        _const_spec((_D, _D)),
        _const_spec((_D, _D)),
        _const_spec((1, _D)),
        _const_spec((_D, _D)),
        _const_spec((1, _D)),
    ],
    out_specs=_row_spec(_D),
    out_shape=jax.ShapeDtypeStruct((_N, _D), jnp.float32),
    compiler_params=pltpu.CompilerParams(dimension_semantics=("arbitrary",)),
)


def kernel(x, edge_index, edge_attr, u, batch,
           W1, b1, W2, b2, Wa, ba, W3, b3, W4, b4):
  del u, batch
  row = edge_index[0].astype(jnp.int32).reshape(_ROWS, 1, _CH)
  pad = jnp.broadcast_to(
      (jnp.arange(_CH, dtype=jnp.int32) % _NDUM) + _N,
      (_RPAD - _ROWS, 1, _CH))
  row = jnp.concatenate([row, pad], axis=0)
  zero = jnp.zeros((_CPY, _ED), jnp.float32)
  ones = jnp.ones((_K * _CH, _ED), jnp.float32)
  acc = _segment_sums(row, edge_attr, zero, ones)
  return _mlp(x, acc[0], acc[1],
              W1[:_D], W1[_D:], b1.reshape(1, _D),
              W2, b2.reshape(1, _D),
              Wa, ba.reshape(1, 1),
              W3[:_D], W3[_D:], b3.reshape(1, _D),
              W4, b4.reshape(1, _D))
